# baseline (speedup 1.0000x reference)
import functools

import jax
import jax.numpy as jnp
from jax import lax
from jax.experimental import pallas as pl
from jax.experimental.pallas import tpu as pltpu
from jax.experimental.pallas import tpu_sc as plsc

N = 10000
E = 320000
CH = 128
NTILE = 16
NW = 2 * NTILE
CPW = 80
E_PAD = NW * CPW * CH
FAKE = E_PAD - E
GIF = 4
RING = 8
STRIPE = 1000
NSTRIPE = N // STRIPE
EPS = 1e-5

_MESH = plsc.VectorSubcoreMesh(core_axis_name="c", subcore_axis_name="s")
_SC_PARAMS = pltpu.CompilerParams(use_tc_tiling_on_sc=False)


def _make_agg(D):
    def body(table_hbm, edges_hbm, out_hbm, acc_sp,
             src_idx, dst_idx, msg, gsem, ssem):
        c = lax.axis_index("c")
        s = lax.axis_index("s")
        w = c * NTILE + s
        r0 = s * STRIPE

        @pl.when(s < NSTRIPE)
        def _():
            pltpu.sync_copy(table_hbm.at[pl.ds(r0, STRIPE)],
                            acc_sp.at[pl.ds(r0, STRIPE)])
        pltpu.sync_copy(edges_hbm.at[0, w], src_idx)
        pltpu.sync_copy(edges_hbm.at[1, w], dst_idx)
        plsc.subcore_barrier()

        def step(j, carry):
            @pl.when(j >= RING)
            def _():
                pltpu.make_async_copy(
                    table_hbm.at[pl.ds(0, CH)], msg.at[0], ssem).wait()

            @pl.when(j < CPW)
            def _():
                pltpu.async_copy(
                    table_hbm.at[src_idx.at[j]],
                    msg.at[lax.rem(j, RING)], gsem)

            @pl.when(j >= GIF)
            def _():
                pltpu.make_async_copy(
                    table_hbm.at[pl.ds(0, CH)], msg.at[0], gsem).wait()
                k = j - GIF
                pltpu.async_copy(
                    msg.at[lax.rem(k, RING)], acc_sp.at[dst_idx.at[k]],
                    ssem, add=True)
            return carry

        lax.fori_loop(0, CPW + GIF, step, 0)
        for _ in range(RING - GIF):
            pltpu.make_async_copy(
                table_hbm.at[pl.ds(0, CH)], msg.at[0], ssem).wait()
        plsc.subcore_barrier()

        @pl.when(s < NSTRIPE)
        def _():
            pltpu.sync_copy(acc_sp.at[pl.ds(r0, STRIPE)],
                            out_hbm.at[c, pl.ds(r0, STRIPE)])

    return pl.kernel(
        body,
        out_type=jax.ShapeDtypeStruct((2, N, D), jnp.float32),
        mesh=_MESH,
        scratch_types=[
            pltpu.VMEM_SHARED((N, D), jnp.float32),
            pltpu.VMEM((CPW, CH), jnp.int32),
            pltpu.VMEM((CPW, CH), jnp.int32),
            pltpu.VMEM((RING, CH, D), jnp.float32),
            pltpu.SemaphoreType.DMA,
            pltpu.SemaphoreType.DMA,
        ],
        compiler_params=_SC_PARAMS,
    )


def _deg_body(edges_hbm, zeros_hbm, ones_hbm, out_hbm, acc_sp, dst_idx,
              ones_v, ssem):
    c = lax.axis_index("c")
    s = lax.axis_index("s")
    w = c * NTILE + s
    r0 = s * STRIPE

    @pl.when(s < NSTRIPE)
    def _():
        pltpu.sync_copy(zeros_hbm.at[pl.ds(r0, STRIPE)],
                        acc_sp.at[pl.ds(r0, STRIPE)])

    pltpu.sync_copy(edges_hbm.at[1, w], dst_idx)
    pltpu.sync_copy(ones_hbm, ones_v)
    plsc.subcore_barrier()

    def fire(j, carry):
        pltpu.async_copy(ones_v, acc_sp.at[dst_idx.at[j]], ssem, add=True)
        return carry

    lax.fori_loop(0, CPW, fire, 0)

    def drain(j, carry):
        pltpu.make_async_copy(ones_hbm, ones_v, ssem).wait()
        return carry

    lax.fori_loop(0, CPW, drain, 0)
    plsc.subcore_barrier()

    @pl.when(s < NSTRIPE)
    def _():
        pltpu.sync_copy(acc_sp.at[pl.ds(r0, STRIPE)],
                        out_hbm.at[c, pl.ds(r0, STRIPE)])


_deg_kernel = pl.kernel(
    _deg_body,
    out_type=jax.ShapeDtypeStruct((2, N, 16), jnp.float32),
    mesh=_MESH,
    scratch_types=[
        pltpu.VMEM_SHARED((N, 16), jnp.float32),
        pltpu.VMEM((CPW, CH), jnp.int32),
        pltpu.VMEM((CH, 16), jnp.float32),
        pltpu.SemaphoreType.DMA,
    ],
    compiler_params=_SC_PARAMS,
)


def _fake_mask():
    iota = lax.broadcasted_iota(jnp.int32, (N, 1), 0)
    return jnp.where(iota < FAKE, 1.0, 0.0)


def _tc1_body(x_ref, w1_ref, counts_ref, h1s_ref, dinv_ref):
    cnt = counts_ref[0] + counts_ref[1] + (1.0 - _fake_mask())
    dinv = lax.rsqrt(cnt)
    h1 = jnp.dot(x_ref[...], w1_ref[...], preferred_element_type=jnp.float32)
    h1s_ref[...] = h1 * dinv
    dinv_ref[...] = dinv


def _bn(u, g, be):
    mean = jnp.mean(u, axis=0, keepdims=True)
    var = jnp.mean((u - mean) * (u - mean), axis=0, keepdims=True)
    return (u - mean) * lax.rsqrt(var + EPS) * g + be


def _tc2_body(acc_ref, h1s_ref, dinv_ref, b1_ref, g1_ref, be1_ref, w2_ref,
              h2s_ref):
    h1s = h1s_ref[...]
    dinv = dinv_ref[...]
    t = acc_ref[0] + acc_ref[1] - h1s - _fake_mask() * h1s[0:1, :]
    u = t * dinv + b1_ref[...]
    r = jnp.maximum(_bn(u, g1_ref[...], be1_ref[...]), 0.0)
    h2 = jnp.dot(r, w2_ref[...], preferred_element_type=jnp.float32)
    h2s_ref[...] = h2 * dinv


def _tc3_body(acc_ref, h2s_ref, dinv_ref, b2_ref, g2_ref, be2_ref, out_ref):
    h2s = h2s_ref[...]
    t = acc_ref[0] + acc_ref[1] - h2s - _fake_mask() * h2s[0:1, :]
    u = t * dinv_ref[...] + b2_ref[...]
    out_ref[...] = _bn(u, g2_ref[...], be2_ref[...])


_agg32 = _make_agg(32)
_agg64 = _make_agg(64)

_tc1 = pl.pallas_call(
    _tc1_body,
    out_shape=[jax.ShapeDtypeStruct((N, 32), jnp.float32),
               jax.ShapeDtypeStruct((N, 1), jnp.float32)],
)
_tc2 = pl.pallas_call(
    _tc2_body,
    out_shape=jax.ShapeDtypeStruct((N, 64), jnp.float32),
)
_tc3 = pl.pallas_call(
    _tc3_body,
    out_shape=jax.ShapeDtypeStruct((N, 64), jnp.float32),
)


@jax.jit
def kernel(x, edge_index, W1, b1, g1, be1, W2, b2, g2, be2):
    ei = edge_index.astype(jnp.int32)
    pad = jnp.stack([jnp.zeros((FAKE,), jnp.int32),
                     jnp.arange(FAKE, dtype=jnp.int32)])
    edges = jnp.reshape(jnp.concatenate([ei, pad], axis=1),
                        (2, NW, CPW, CH))
    zeros = jnp.zeros((N, 16), jnp.float32)
    ones = jnp.ones((CH, 16), jnp.float32)
    counts = _deg_kernel(edges, zeros, ones)
    h1s, dinv = _tc1(x, W1, counts[:, :, :1])
    acc1 = _agg32(h1s, edges)
    h2s = _tc2(acc1, h1s, dinv, b1, g1, be1, W2)
    acc2 = _agg64(h2s, edges)
    return _tc3(acc2, h2s, dinv, b2, g2, be2)


# speedup vs baseline: 2.3019x; 2.3019x over previous
import functools

import jax
import jax.numpy as jnp
from jax import lax
from jax.experimental import pallas as pl
from jax.experimental.pallas import tpu as pltpu
from jax.experimental.pallas import tpu_sc as plsc

N = 10000
E = 320000
CH = 128
NTILE = 16
NW = 2 * NTILE
CPW = 80
E_PAD = NW * CPW * CH
FAKE = E_PAD - E
GIF = 4
RING = 8
STRIPE = 1000
NSTRIPE = N // STRIPE
EPS = 1e-5

_MESH = plsc.VectorSubcoreMesh(core_axis_name="c", subcore_axis_name="s")
_SC_PARAMS = pltpu.CompilerParams(use_tc_tiling_on_sc=False)


def _make_agg(D):
    def body(table_hbm, edges_hbm, out_hbm, acc_sp,
             src_idx, dst_idx, msg, gsem, ssem):
        c = lax.axis_index("c")
        s = lax.axis_index("s")
        w = c * NTILE + s
        r0 = s * STRIPE

        @pl.when(s < NSTRIPE)
        def _():
            pltpu.sync_copy(table_hbm.at[pl.ds(r0, STRIPE)],
                            acc_sp.at[pl.ds(r0, STRIPE)])
        pltpu.sync_copy(edges_hbm.at[0, w], src_idx)
        pltpu.sync_copy(edges_hbm.at[1, w], dst_idx)
        plsc.subcore_barrier()

        def step(j, carry):
            @pl.when(j >= RING)
            def _():
                pltpu.make_async_copy(
                    table_hbm.at[pl.ds(0, CH)], msg.at[0], ssem).wait()

            @pl.when(j < CPW)
            def _():
                pltpu.async_copy(
                    table_hbm.at[src_idx.at[j]],
                    msg.at[lax.rem(j, RING)], gsem)

            @pl.when(j >= GIF)
            def _():
                pltpu.make_async_copy(
                    table_hbm.at[pl.ds(0, CH)], msg.at[0], gsem).wait()
                k = j - GIF
                pltpu.async_copy(
                    msg.at[lax.rem(k, RING)], acc_sp.at[dst_idx.at[k]],
                    ssem, add=True)
            return carry

        lax.fori_loop(0, CPW + GIF, step, 0)
        for _ in range(RING - GIF):
            pltpu.make_async_copy(
                table_hbm.at[pl.ds(0, CH)], msg.at[0], ssem).wait()
        plsc.subcore_barrier()

        @pl.when(s < NSTRIPE)
        def _():
            pltpu.sync_copy(acc_sp.at[pl.ds(r0, STRIPE)],
                            out_hbm.at[c, pl.ds(r0, STRIPE)])

    return pl.kernel(
        body,
        out_type=jax.ShapeDtypeStruct((2, N, D), jnp.float32),
        mesh=_MESH,
        scratch_types=[
            pltpu.VMEM_SHARED((N, D), jnp.float32),
            pltpu.VMEM((CPW, CH), jnp.int32),
            pltpu.VMEM((CPW, CH), jnp.int32),
            pltpu.VMEM((RING, CH, D), jnp.float32),
            pltpu.SemaphoreType.DMA,
            pltpu.SemaphoreType.DMA,
        ],
        compiler_params=_SC_PARAMS,
    )


def _deg_body(edges_hbm, zeros_hbm, ones_hbm, out_hbm, acc_sp, dst_idx,
              ones_v, ssem):
    c = lax.axis_index("c")
    s = lax.axis_index("s")
    w = c * NTILE + s
    r0 = s * STRIPE

    @pl.when(s < NSTRIPE)
    def _():
        pltpu.sync_copy(zeros_hbm.at[pl.ds(r0, STRIPE)],
                        acc_sp.at[pl.ds(r0, STRIPE)])

    pltpu.sync_copy(edges_hbm.at[1, w], dst_idx)
    pltpu.sync_copy(ones_hbm, ones_v)
    plsc.subcore_barrier()

    def fire(j, carry):
        pltpu.async_copy(ones_v, acc_sp.at[dst_idx.at[j]], ssem, add=True)
        return carry

    lax.fori_loop(0, CPW, fire, 0)

    def drain(j, carry):
        pltpu.make_async_copy(ones_hbm, ones_v, ssem).wait()
        return carry

    lax.fori_loop(0, CPW, drain, 0)
    plsc.subcore_barrier()

    @pl.when(s < NSTRIPE)
    def _():
        pltpu.sync_copy(acc_sp.at[pl.ds(r0, STRIPE)],
                        out_hbm.at[c, pl.ds(r0, STRIPE)])


_deg_kernel = pl.kernel(
    _deg_body,
    out_type=jax.ShapeDtypeStruct((2, N, 16), jnp.float32),
    mesh=_MESH,
    scratch_types=[
        pltpu.VMEM_SHARED((N, 16), jnp.float32),
        pltpu.VMEM((CPW, CH), jnp.int32),
        pltpu.VMEM((CH, 16), jnp.float32),
        pltpu.SemaphoreType.DMA,
    ],
    compiler_params=_SC_PARAMS,
)


def _fake_mask():
    iota = lax.broadcasted_iota(jnp.int32, (N, 1), 0)
    return jnp.where(iota < FAKE, 1.0, 0.0)


def _tc1_body(x_ref, w1_ref, counts_ref, h1s_ref, dinv_ref):
    cnt = counts_ref[0] + counts_ref[1] + (1.0 - _fake_mask())
    dinv = lax.rsqrt(cnt)
    h1 = jnp.dot(x_ref[...], w1_ref[...], preferred_element_type=jnp.float32)
    h1s_ref[...] = h1 * dinv
    dinv_ref[...] = dinv


def _bn(u, g, be):
    mean = jnp.mean(u, axis=0, keepdims=True)
    var = jnp.mean((u - mean) * (u - mean), axis=0, keepdims=True)
    return (u - mean) * lax.rsqrt(var + EPS) * g + be


def _tc2_body(acc_ref, h1s_ref, dinv_ref, b1_ref, g1_ref, be1_ref, w2_ref,
              h2s_ref):
    h1s = h1s_ref[...]
    dinv = dinv_ref[...]
    t = acc_ref[0] + acc_ref[1] - (1.0 + _fake_mask()) * h1s
    u = t * dinv + b1_ref[...]
    r = jnp.maximum(_bn(u, g1_ref[...], be1_ref[...]), 0.0)
    h2 = jnp.dot(r, w2_ref[...], preferred_element_type=jnp.float32)
    h2s_ref[...] = h2 * dinv


def _tc3_body(acc_ref, h2s_ref, dinv_ref, b2_ref, g2_ref, be2_ref, out_ref):
    h2s = h2s_ref[...]
    t = acc_ref[0] + acc_ref[1] - (1.0 + _fake_mask()) * h2s
    u = t * dinv_ref[...] + b2_ref[...]
    out_ref[...] = _bn(u, g2_ref[...], be2_ref[...])


_agg32 = _make_agg(32)
_agg64 = _make_agg(64)

_tc1 = pl.pallas_call(
    _tc1_body,
    out_shape=[jax.ShapeDtypeStruct((N, 32), jnp.float32),
               jax.ShapeDtypeStruct((N, 1), jnp.float32)],
)
_tc2 = pl.pallas_call(
    _tc2_body,
    out_shape=jax.ShapeDtypeStruct((N, 64), jnp.float32),
)
_tc3 = pl.pallas_call(
    _tc3_body,
    out_shape=jax.ShapeDtypeStruct((N, 64), jnp.float32),
)


@jax.jit
def kernel(x, edge_index, W1, b1, g1, be1, W2, b2, g2, be2):
    ei = edge_index.astype(jnp.int32)
    fake_ids = jnp.arange(FAKE, dtype=jnp.int32)
    pad = jnp.stack([fake_ids, fake_ids])
    edges = jnp.reshape(jnp.concatenate([ei, pad], axis=1),
                        (2, NW, CPW, CH))
    zeros = jnp.zeros((N, 16), jnp.float32)
    ones = jnp.ones((CH, 16), jnp.float32)
    counts = _deg_kernel(edges, zeros, ones)
    h1s, dinv = _tc1(x, W1, counts[:, :, :1])
    acc1 = _agg32(h1s, edges)
    h2s = _tc2(acc1, h1s, dinv, b1, g1, be1, W2)
    acc2 = _agg64(h2s, edges)
    return _tc3(acc2, h2s, dinv, b2, g2, be2)


# speedup vs baseline: 2.3075x; 1.0024x over previous
import functools

import jax
import jax.numpy as jnp
from jax import lax
from jax.experimental import pallas as pl
from jax.experimental.pallas import tpu as pltpu
from jax.experimental.pallas import tpu_sc as plsc

N = 10000
E = 320000
CH = 128
NTILE = 16
NW = 2 * NTILE
CPW = 80
E_PAD = NW * CPW * CH
FAKE = E_PAD - E
STRIPE = 1000
NSTRIPE = N // STRIPE
EPS = 1e-5

_MESH = plsc.VectorSubcoreMesh(core_axis_name="c", subcore_axis_name="s")
_SC_PARAMS = pltpu.CompilerParams(use_tc_tiling_on_sc=False)


def _make_agg(D, GIF, RING):
    def body(table_hbm, edges_hbm, out_hbm, acc_sp,
             src_idx, dst_idx, msg, gsem, ssem):
        c = lax.axis_index("c")
        s = lax.axis_index("s")
        w = c * NTILE + s
        r0 = s * STRIPE

        @pl.when(s < NSTRIPE)
        def _():
            pltpu.sync_copy(table_hbm.at[pl.ds(r0, STRIPE)],
                            acc_sp.at[pl.ds(r0, STRIPE)])
        pltpu.sync_copy(edges_hbm.at[0, w], src_idx)
        pltpu.sync_copy(edges_hbm.at[1, w], dst_idx)
        plsc.subcore_barrier()

        def step(j, carry):
            @pl.when(j >= RING)
            def _():
                pltpu.make_async_copy(
                    table_hbm.at[pl.ds(0, CH)], msg.at[0], ssem).wait()

            @pl.when(j < CPW)
            def _():
                pltpu.async_copy(
                    table_hbm.at[src_idx.at[j]],
                    msg.at[lax.rem(j, RING)], gsem)

            @pl.when(j >= GIF)
            def _():
                pltpu.make_async_copy(
                    table_hbm.at[pl.ds(0, CH)], msg.at[0], gsem).wait()
                k = j - GIF
                pltpu.async_copy(
                    msg.at[lax.rem(k, RING)], acc_sp.at[dst_idx.at[k]],
                    ssem, add=True)
            return carry

        lax.fori_loop(0, CPW + GIF, step, 0)
        for _ in range(RING - GIF):
            pltpu.make_async_copy(
                table_hbm.at[pl.ds(0, CH)], msg.at[0], ssem).wait()
        plsc.subcore_barrier()

        @pl.when(s < NSTRIPE)
        def _():
            pltpu.sync_copy(acc_sp.at[pl.ds(r0, STRIPE)],
                            out_hbm.at[c, pl.ds(r0, STRIPE)])

    return pl.kernel(
        body,
        out_type=jax.ShapeDtypeStruct((2, N, D), jnp.float32),
        mesh=_MESH,
        scratch_types=[
            pltpu.VMEM_SHARED((N, D), jnp.float32),
            pltpu.VMEM((CPW, CH), jnp.int32),
            pltpu.VMEM((CPW, CH), jnp.int32),
            pltpu.VMEM((RING, CH, D), jnp.float32),
            pltpu.SemaphoreType.DMA,
            pltpu.SemaphoreType.DMA,
        ],
        compiler_params=_SC_PARAMS,
    )


def _deg_body(edges_hbm, zeros_hbm, ones_hbm, out_hbm, acc_sp, dst_idx,
              ones_v, ssem):
    c = lax.axis_index("c")
    s = lax.axis_index("s")
    w = c * NTILE + s
    r0 = s * STRIPE

    @pl.when(s < NSTRIPE)
    def _():
        pltpu.sync_copy(zeros_hbm.at[pl.ds(r0, STRIPE)],
                        acc_sp.at[pl.ds(r0, STRIPE)])

    pltpu.sync_copy(edges_hbm.at[1, w], dst_idx)
    pltpu.sync_copy(ones_hbm, ones_v)
    plsc.subcore_barrier()

    def fire(j, carry):
        pltpu.async_copy(ones_v, acc_sp.at[dst_idx.at[j]], ssem, add=True)
        return carry

    lax.fori_loop(0, CPW, fire, 0)

    def drain(j, carry):
        pltpu.make_async_copy(ones_hbm, ones_v, ssem).wait()
        return carry

    lax.fori_loop(0, CPW, drain, 0)
    plsc.subcore_barrier()

    @pl.when(s < NSTRIPE)
    def _():
        pltpu.sync_copy(acc_sp.at[pl.ds(r0, STRIPE)],
                        out_hbm.at[c, pl.ds(r0, STRIPE)])


_deg_kernel = pl.kernel(
    _deg_body,
    out_type=jax.ShapeDtypeStruct((2, N, 16), jnp.float32),
    mesh=_MESH,
    scratch_types=[
        pltpu.VMEM_SHARED((N, 16), jnp.float32),
        pltpu.VMEM((CPW, CH), jnp.int32),
        pltpu.VMEM((CH, 16), jnp.float32),
        pltpu.SemaphoreType.DMA,
    ],
    compiler_params=_SC_PARAMS,
)


def _fake_mask():
    iota = lax.broadcasted_iota(jnp.int32, (N, 1), 0)
    return jnp.where(iota < FAKE, 1.0, 0.0)


def _tc0_body(x_ref, w1_ref, h1_ref):
    h1_ref[...] = jnp.dot(x_ref[...], w1_ref[...],
                          preferred_element_type=jnp.float32)


def _tc1_body(h1_ref, counts_ref, h1s_ref, dinv_ref):
    cnt = counts_ref[0] + counts_ref[1] + (1.0 - _fake_mask())
    dinv = lax.rsqrt(cnt)
    h1s_ref[...] = h1_ref[...] * dinv
    dinv_ref[...] = dinv


def _bn(u, g, be):
    ones_row = jnp.full((1, N), 1.0 / N, jnp.float32)
    mean = jnp.dot(ones_row, u, preferred_element_type=jnp.float32)
    ex2 = jnp.dot(ones_row, u * u, preferred_element_type=jnp.float32)
    var = ex2 - mean * mean
    return (u - mean) * lax.rsqrt(var + EPS) * g + be


def _tc2_body(acc_ref, h1s_ref, dinv_ref, b1_ref, g1_ref, be1_ref, w2_ref,
              h2s_ref):
    h1s = h1s_ref[...]
    dinv = dinv_ref[...]
    t = acc_ref[0] + acc_ref[1] - (1.0 + _fake_mask()) * h1s
    u = t * dinv + b1_ref[...]
    r = jnp.maximum(_bn(u, g1_ref[...], be1_ref[...]), 0.0)
    h2 = jnp.dot(r, w2_ref[...], preferred_element_type=jnp.float32)
    h2s_ref[...] = h2 * dinv


def _tc3_body(acc_ref, h2s_ref, dinv_ref, b2_ref, g2_ref, be2_ref, out_ref):
    h2s = h2s_ref[...]
    t = acc_ref[0] + acc_ref[1] - (1.0 + _fake_mask()) * h2s
    u = t * dinv_ref[...] + b2_ref[...]
    out_ref[...] = _bn(u, g2_ref[...], be2_ref[...])


_agg32 = _make_agg(32, 6, 12)
_agg64 = _make_agg(64, 4, 8)

_tc0 = pl.pallas_call(
    _tc0_body,
    out_shape=jax.ShapeDtypeStruct((N, 32), jnp.float32),
)
_tc1 = pl.pallas_call(
    _tc1_body,
    out_shape=[jax.ShapeDtypeStruct((N, 32), jnp.float32),
               jax.ShapeDtypeStruct((N, 1), jnp.float32)],
)
_tc2 = pl.pallas_call(
    _tc2_body,
    out_shape=jax.ShapeDtypeStruct((N, 64), jnp.float32),
)
_tc3 = pl.pallas_call(
    _tc3_body,
    out_shape=jax.ShapeDtypeStruct((N, 64), jnp.float32),
)


@jax.jit
def kernel(x, edge_index, W1, b1, g1, be1, W2, b2, g2, be2):
    ei = edge_index.astype(jnp.int32)
    fake_ids = jnp.arange(FAKE, dtype=jnp.int32)
    pad = jnp.stack([fake_ids, fake_ids])
    edges = jnp.reshape(jnp.concatenate([ei, pad], axis=1),
                        (2, NW, CPW, CH))
    zeros = jnp.zeros((N, 16), jnp.float32)
    ones = jnp.ones((CH, 16), jnp.float32)
    counts = _deg_kernel(edges, zeros, ones)
    h1 = _tc0(x, W1)
    h1s, dinv = _tc1(h1, counts[:, :, :1])
    acc1 = _agg32(h1s, edges)
    h2s = _tc2(acc1, h1s, dinv, b1, g1, be1, W2)
    acc2 = _agg64(h2s, edges)
    return _tc3(acc2, h2s, dinv, b2, g2, be2)


# speedup vs baseline: 2.3109x; 1.0015x over previous
import functools

import jax
import jax.numpy as jnp
from jax import lax
from jax.experimental import pallas as pl
from jax.experimental.pallas import tpu as pltpu
from jax.experimental.pallas import tpu_sc as plsc

N = 10000
E = 320000
CH = 128
NTILE = 16
NW = 2 * NTILE
CPW = 80
E_PAD = NW * CPW * CH
FAKE = E_PAD - E
STRIPE = 1000
NSTRIPE = N // STRIPE
EPS = 1e-5

_MESH = plsc.VectorSubcoreMesh(core_axis_name="c", subcore_axis_name="s")
_SC_PARAMS = pltpu.CompilerParams(use_tc_tiling_on_sc=False)


def _make_agg(D, GIF, RING):
    def body(table_hbm, edges_hbm, out_hbm, acc_sp,
             src_idx, dst_idx, msg, gsem, ssem):
        c = lax.axis_index("c")
        s = lax.axis_index("s")
        w = c * NTILE + s
        r0 = s * STRIPE

        @pl.when(s < NSTRIPE)
        def _():
            pltpu.sync_copy(table_hbm.at[pl.ds(r0, STRIPE)],
                            acc_sp.at[pl.ds(r0, STRIPE)])
        pltpu.sync_copy(edges_hbm.at[0, w], src_idx)
        pltpu.sync_copy(edges_hbm.at[1, w], dst_idx)
        plsc.subcore_barrier()

        def step(j, carry):
            @pl.when(j >= RING)
            def _():
                pltpu.make_async_copy(
                    table_hbm.at[pl.ds(0, CH)], msg.at[0], ssem).wait()

            @pl.when(j < CPW)
            def _():
                pltpu.async_copy(
                    table_hbm.at[src_idx.at[j]],
                    msg.at[lax.rem(j, RING)], gsem)

            @pl.when(j >= GIF)
            def _():
                pltpu.make_async_copy(
                    table_hbm.at[pl.ds(0, CH)], msg.at[0], gsem).wait()
                k = j - GIF
                pltpu.async_copy(
                    msg.at[lax.rem(k, RING)], acc_sp.at[dst_idx.at[k]],
                    ssem, add=True)
            return carry

        lax.fori_loop(0, CPW + GIF, step, 0)
        for _ in range(RING - GIF):
            pltpu.make_async_copy(
                table_hbm.at[pl.ds(0, CH)], msg.at[0], ssem).wait()
        plsc.subcore_barrier()

        @pl.when(s < NSTRIPE)
        def _():
            pltpu.sync_copy(acc_sp.at[pl.ds(r0, STRIPE)],
                            out_hbm.at[c, pl.ds(r0, STRIPE)])

    return pl.kernel(
        body,
        out_type=jax.ShapeDtypeStruct((2, N, D), jnp.float32),
        mesh=_MESH,
        scratch_types=[
            pltpu.VMEM_SHARED((N, D), jnp.float32),
            pltpu.VMEM((CPW, CH), jnp.int32),
            pltpu.VMEM((CPW, CH), jnp.int32),
            pltpu.VMEM((RING, CH, D), jnp.float32),
            pltpu.SemaphoreType.DMA,
            pltpu.SemaphoreType.DMA,
        ],
        compiler_params=_SC_PARAMS,
    )


def _deg_body(edges_hbm, zeros_hbm, ones_hbm, out_hbm, acc_sp, dst_idx,
              ones_v, ssem):
    c = lax.axis_index("c")
    s = lax.axis_index("s")
    w = c * NTILE + s
    r0 = s * STRIPE

    @pl.when(s < NSTRIPE)
    def _():
        pltpu.sync_copy(zeros_hbm.at[pl.ds(r0, STRIPE)],
                        acc_sp.at[pl.ds(r0, STRIPE)])

    pltpu.sync_copy(edges_hbm.at[1, w], dst_idx)
    pltpu.sync_copy(ones_hbm, ones_v)
    plsc.subcore_barrier()

    def fire(j, carry):
        pltpu.async_copy(ones_v, acc_sp.at[dst_idx.at[j]], ssem, add=True)
        return carry

    lax.fori_loop(0, CPW, fire, 0)

    def drain(j, carry):
        pltpu.make_async_copy(ones_hbm, ones_v, ssem).wait()
        return carry

    lax.fori_loop(0, CPW, drain, 0)
    plsc.subcore_barrier()

    @pl.when(s < NSTRIPE)
    def _():
        pltpu.sync_copy(acc_sp.at[pl.ds(r0, STRIPE)],
                        out_hbm.at[c, pl.ds(r0, STRIPE)])


_deg_kernel = pl.kernel(
    _deg_body,
    out_type=jax.ShapeDtypeStruct((2, N, 16), jnp.float32),
    mesh=_MESH,
    scratch_types=[
        pltpu.VMEM_SHARED((N, 16), jnp.float32),
        pltpu.VMEM((CPW, CH), jnp.int32),
        pltpu.VMEM((CH, 16), jnp.float32),
        pltpu.SemaphoreType.DMA,
    ],
    compiler_params=_SC_PARAMS,
)


def _fake_mask():
    iota = lax.broadcasted_iota(jnp.int32, (N, 1), 0)
    return jnp.where(iota < FAKE, 1.0, 0.0)


def _tc0_body(x_ref, w1_ref, h1_ref):
    h1_ref[...] = jnp.dot(x_ref[...], w1_ref[...],
                          preferred_element_type=jnp.float32)


def _tc1_body(h1_ref, counts_ref, h1s_ref, dinv_ref):
    cnt = counts_ref[0] + counts_ref[1] + (1.0 - _fake_mask())
    dinv = lax.rsqrt(cnt)
    h1s_ref[...] = h1_ref[...] * dinv
    dinv_ref[...] = dinv


def _bn(u, g, be):
    ones_row = jnp.full((1, N), 1.0 / N, jnp.float32)
    mean = jnp.dot(ones_row, u, preferred_element_type=jnp.float32)
    d = u - mean
    var = jnp.dot(ones_row, d * d, preferred_element_type=jnp.float32)
    return d * lax.rsqrt(var + EPS) * g + be


def _tc2_body(acc_ref, h1s_ref, dinv_ref, b1_ref, g1_ref, be1_ref, w2_ref,
              h2s_ref):
    h1s = h1s_ref[...]
    dinv = dinv_ref[...]
    t = acc_ref[0] + acc_ref[1] - (1.0 + _fake_mask()) * h1s
    u = t * dinv + b1_ref[...]
    r = jnp.maximum(_bn(u, g1_ref[...], be1_ref[...]), 0.0)
    h2 = jnp.dot(r, w2_ref[...], preferred_element_type=jnp.float32)
    h2s_ref[...] = h2 * dinv


def _tc3_body(acc_ref, h2s_ref, dinv_ref, b2_ref, g2_ref, be2_ref, out_ref):
    h2s = h2s_ref[...]
    t = acc_ref[0] + acc_ref[1] - (1.0 + _fake_mask()) * h2s
    u = t * dinv_ref[...] + b2_ref[...]
    out_ref[...] = _bn(u, g2_ref[...], be2_ref[...])


_agg32 = _make_agg(32, 6, 12)
_agg64 = _make_agg(64, 4, 8)

_tc0 = pl.pallas_call(
    _tc0_body,
    out_shape=jax.ShapeDtypeStruct((N, 32), jnp.float32),
)
_tc1 = pl.pallas_call(
    _tc1_body,
    out_shape=[jax.ShapeDtypeStruct((N, 32), jnp.float32),
               jax.ShapeDtypeStruct((N, 1), jnp.float32)],
)
_tc2 = pl.pallas_call(
    _tc2_body,
    out_shape=jax.ShapeDtypeStruct((N, 64), jnp.float32),
)
_tc3 = pl.pallas_call(
    _tc3_body,
    out_shape=jax.ShapeDtypeStruct((N, 64), jnp.float32),
)


@jax.jit
def kernel(x, edge_index, W1, b1, g1, be1, W2, b2, g2, be2):
    ei = edge_index.astype(jnp.int32)
    fake_ids = jnp.arange(FAKE, dtype=jnp.int32)
    pad = jnp.stack([fake_ids, fake_ids])
    edges = jnp.reshape(jnp.concatenate([ei, pad], axis=1),
                        (2, NW, CPW, CH))
    zeros = jnp.zeros((N, 16), jnp.float32)
    ones = jnp.ones((CH, 16), jnp.float32)
    counts = _deg_kernel(edges, zeros, ones)
    h1 = _tc0(x, W1)
    h1s, dinv = _tc1(h1, counts[:, :, :1])
    acc1 = _agg32(h1s, edges)
    h2s = _tc2(acc1, h1s, dinv, b1, g1, be1, W2)
    acc2 = _agg64(h2s, edges)
    return _tc3(acc2, h2s, dinv, b2, g2, be2)
